# baseline (speedup 1.0000x reference)
import functools

import jax
import jax.numpy as jnp
from jax.experimental import pallas as pl

VOCAB = 8192
HALF = VOCAB // 2
D = 3
BETA = 0.25


def _half_argmin(scores, base):
    m = jnp.min(scores, axis=1, keepdims=True)
    iota = jax.lax.broadcasted_iota(jnp.int32, scores.shape, 1)
    am = jnp.min(jnp.where(scores == m, iota, VOCAB), axis=1, keepdims=True)
    return m, am + base


def _vq_body(flat_ref, cbt_ref, cb_ref, q_ref, idx_ref, acc_ref):
    i = pl.program_id(0)
    f = flat_ref[...]
    cbt = cbt_ref[...]
    fnorm = jnp.sum(f * f, axis=1, keepdims=True)
    cnorm = jnp.sum(cbt * cbt, axis=0, keepdims=True)
    dot = jax.lax.dot_general(
        f.astype(jnp.bfloat16), cbt.astype(jnp.bfloat16),
        (((1,), (0,)), ((), ())),
        preferred_element_type=jnp.float32)
    scores = (fnorm - 2.0 * dot) + cnorm
    m1, am1 = _half_argmin(scores[:, :HALF], 0)
    m2, am2 = _half_argmin(scores[:, HALF:], HALF)
    m1q = m1.astype(jnp.bfloat16).astype(jnp.float32)
    upd = m2 < m1q
    am = jnp.where(upd, am2, am1)
    idx_ref[...] = am
    iota = jax.lax.broadcasted_iota(jnp.int32, scores.shape, 1)
    onehot = (iota == am).astype(jnp.float32)
    quant = jax.lax.dot_general(
        onehot, cb_ref[...], (((1,), (0,)), ((), ())),
        preferred_element_type=jnp.float32,
        precision=jax.lax.Precision.HIGHEST)
    diff = quant - f
    q_ref[...] = f + diff

    @pl.when(i == 0)
    def _():
        acc_ref[...] = jnp.zeros_like(acc_ref)

    acc_ref[...] += jnp.sum(diff * diff).reshape(1, 1)


@functools.partial(jax.jit, static_argnames=("block_t",))
def _vq(flat, codebook, block_t=256):
    n = flat.shape[0]
    grid = n // block_t
    q, idx, acc = pl.pallas_call(
        _vq_body,
        grid=(grid,),
        in_specs=[
            pl.BlockSpec((block_t, D), lambda i: (i, 0)),
            pl.BlockSpec((D, VOCAB), lambda i: (0, 0)),
            pl.BlockSpec((VOCAB, D), lambda i: (0, 0)),
        ],
        out_specs=[
            pl.BlockSpec((block_t, D), lambda i: (i, 0)),
            pl.BlockSpec((block_t, 1), lambda i: (i, 0)),
            pl.BlockSpec((1, 1), lambda i: (0, 0)),
        ],
        out_shape=[
            jax.ShapeDtypeStruct((n, D), jnp.float32),
            jax.ShapeDtypeStruct((n, 1), jnp.int32),
            jax.ShapeDtypeStruct((1, 1), jnp.float32),
        ],
    )(flat, codebook.T, codebook)
    return q, idx, acc


def kernel(feats, codebook):
    b, l, d = feats.shape
    flat = feats.reshape(-1, d)
    q, idx, acc = _vq(flat, codebook)
    n = b * l
    vq_loss = (1.0 + BETA) * (acc[0, 0] / jnp.float32(n * d))
    return q.reshape(b, l, d), idx.reshape(b, l), vq_loss
